# baseline (speedup 1.0000x reference)
import jax
import jax.numpy as jnp
from jax import lax
from jax.experimental import pallas as pl
from jax.experimental.pallas import tpu as pltpu
from jax.experimental.pallas import tpu_sc as plsc

N = 10000
E = 160000
D = 256
H = 128
NP = 10240
EP = 163840
NTILES = 32
CHUNKS = 40
CW = 128
AGG_CHUNKS = 80
ACW = 128
SLABS = 10
SLAB_CHUNKS = AGG_CHUNKS // SLABS
STRIPE = NP // 16

_mesh = plsc.VectorSubcoreMesh(core_axis_name="c", subcore_axis_name="s")


def _deg_kernel(dst3, degp, idx_v, ones_v, buf_v, deg_sh, sem):
    c = lax.axis_index("c")
    s = lax.axis_index("s")
    w = s * 2 + c

    @pl.loop(0, STRIPE, step=16)
    def _(i):
        buf_v[pl.ds(i, 16)] = jnp.zeros((16,), jnp.float32)

    @pl.loop(0, CW, step=16)
    def _(i):
        ones_v[pl.ds(i, 16)] = jnp.ones((16,), jnp.float32)

    pltpu.sync_copy(buf_v, deg_sh.at[pl.ds(s * STRIPE, STRIPE)])
    pltpu.sync_copy(dst3.at[w], idx_v)
    plsc.subcore_barrier()

    @pl.loop(0, CHUNKS)
    def _(j):
        pltpu.sync_copy(ones_v, deg_sh.at[idx_v.at[j]], add=True)

    plsc.subcore_barrier()
    pltpu.sync_copy(deg_sh.at[pl.ds(s * STRIPE, STRIPE)],
                    degp.at[c, pl.ds(s * STRIPE, STRIPE)])


@jax.jit
def _deg_call(dst3):
    k = pl.kernel(
        _deg_kernel,
        out_type=jax.ShapeDtypeStruct((2, NP), jnp.float32),
        mesh=_mesh,
        scratch_types=[
            pltpu.VMEM((CHUNKS, CW), jnp.int32),
            pltpu.VMEM((CW,), jnp.float32),
            pltpu.VMEM((STRIPE,), jnp.float32),
            pltpu.VMEM_SHARED((NP,), jnp.float32),
            pltpu.SemaphoreType.DMA,
        ],
    )
    return k(dst3)


def _agg_kernel(h0, h1, src3, dst3, agg0, agg1,
                srcv, dstv, rows, zbuf, agg_sh, sem, sem2):
    c = lax.axis_index("c")
    s = lax.axis_index("s")

    @pl.loop(0, 8)
    def _(r):
        @pl.loop(0, H, step=16)
        def _(l2):
            zbuf[r, pl.ds(l2, 16)] = jnp.zeros((16,), jnp.float32)

    @pl.loop(0, STRIPE, step=8)
    def _(r):
        pltpu.sync_copy(zbuf, agg_sh.at[pl.ds(s * STRIPE + r, 8)])

    plsc.subcore_barrier()

    def run(h_half, out_half):
        @pl.loop(0, SLABS)
        def _(t):
            pltpu.sync_copy(src3.at[s, pl.ds(t * SLAB_CHUNKS, SLAB_CHUNKS)],
                            srcv)
            pltpu.sync_copy(dst3.at[s, pl.ds(t * SLAB_CHUNKS, SLAB_CHUNKS)],
                            dstv)

            @pl.loop(0, SLAB_CHUNKS + 1)
            def _(j):
                @pl.when(j < SLAB_CHUNKS)
                def _():
                    @pl.when(j >= 2)
                    def _():
                        pltpu.make_async_copy(rows.at[0],
                                              agg_sh.at[pl.ds(0, ACW)],
                                              sem2).wait()

                    pltpu.async_copy(h_half.at[srcv.at[j]],
                                     rows.at[lax.rem(j, 2)], sem)

                @pl.when(j > 0)
                def _():
                    pltpu.make_async_copy(h_half.at[pl.ds(0, ACW)],
                                          rows.at[0], sem).wait()
                    pltpu.async_copy(rows.at[lax.rem(j + 1, 2)],
                                     agg_sh.at[dstv.at[j - 1]], sem2,
                                     add=True)

            pltpu.make_async_copy(rows.at[0], agg_sh.at[pl.ds(0, ACW)],
                                  sem2).wait()
            pltpu.make_async_copy(rows.at[0], agg_sh.at[pl.ds(0, ACW)],
                                  sem2).wait()

        plsc.subcore_barrier()
        pltpu.sync_copy(agg_sh.at[pl.ds(s * STRIPE, STRIPE)],
                        out_half.at[pl.ds(s * STRIPE, STRIPE)])

    @pl.when(c == 0)
    def _():
        run(h0, agg0)

    @pl.when(c == 1)
    def _():
        run(h1, agg1)


@jax.jit
def _agg_call(h0, h1, src3, dst3):
    k = pl.kernel(
        _agg_kernel,
        out_type=(jax.ShapeDtypeStruct((NP, H), jnp.float32),
                  jax.ShapeDtypeStruct((NP, H), jnp.float32)),
        mesh=_mesh,
        scratch_types=[
            pltpu.VMEM((SLAB_CHUNKS, ACW), jnp.int32),
            pltpu.VMEM((SLAB_CHUNKS, ACW), jnp.int32),
            pltpu.VMEM((2, ACW, H), jnp.float32),
            pltpu.VMEM((8, H), jnp.float32),
            pltpu.VMEM_SHARED((NP, H), jnp.float32),
            pltpu.SemaphoreType.DMA,
            pltpu.SemaphoreType.DMA,
        ],
    )
    return k(h0, h1, src3, dst3)


def _norm_kernel(deg_ref, feat_ref, h0_ref, h1_ref, norm_ref):
    deg = deg_ref[...]
    norm = jnp.where(deg > 0.0,
                     lax.rsqrt(jnp.where(deg > 0.0, deg, 1.0)), 0.0)
    h = feat_ref[...] * norm
    h0_ref[...] = h[:, :H]
    h1_ref[...] = h[:, H:]
    norm_ref[...] = norm


@jax.jit
def _norm_call(deg_col, feat):
    blk = 1000
    return pl.pallas_call(
        _norm_kernel,
        grid=(N // blk,),
        in_specs=[
            pl.BlockSpec((blk, 1), lambda i: (i, 0)),
            pl.BlockSpec((blk, D), lambda i: (i, 0)),
        ],
        out_specs=[
            pl.BlockSpec((blk, H), lambda i: (i, 0)),
            pl.BlockSpec((blk, H), lambda i: (i, 0)),
            pl.BlockSpec((blk, 1), lambda i: (i, 0)),
        ],
        out_shape=[
            jax.ShapeDtypeStruct((N, H), jnp.float32),
            jax.ShapeDtypeStruct((N, H), jnp.float32),
            jax.ShapeDtypeStruct((N, 1), jnp.float32),
        ],
    )(deg_col, feat)


def _out_kernel(a0_ref, a1_ref, n_ref, w0_ref, w1_ref, b_ref, o_ref):
    r = jnp.dot(a0_ref[...], w0_ref[...],
                preferred_element_type=jnp.float32,
                precision=lax.Precision.HIGHEST)
    r = r + jnp.dot(a1_ref[...], w1_ref[...],
                    preferred_element_type=jnp.float32,
                    precision=lax.Precision.HIGHEST)
    o_ref[...] = r * n_ref[...] + b_ref[...]


@jax.jit
def _out_call(agg0, agg1, normc, w0, w1, bias2):
    blk = 1000
    return pl.pallas_call(
        _out_kernel,
        grid=(N // blk,),
        in_specs=[
            pl.BlockSpec((blk, H), lambda i: (i, 0)),
            pl.BlockSpec((blk, H), lambda i: (i, 0)),
            pl.BlockSpec((blk, 1), lambda i: (i, 0)),
            pl.BlockSpec((H, D), lambda i: (0, 0)),
            pl.BlockSpec((H, D), lambda i: (0, 0)),
            pl.BlockSpec((1, D), lambda i: (0, 0)),
        ],
        out_specs=pl.BlockSpec((blk, D), lambda i: (i, 0)),
        out_shape=jax.ShapeDtypeStruct((N, D), jnp.float32),
    )(agg0, agg1, normc, w0, w1, bias2)


def kernel(feat, edge_index, weight, bias):
    src = edge_index[0]
    dst = edge_index[1]
    padlen = EP - E
    src_p = jnp.concatenate([src, jnp.zeros((padlen,), jnp.int32)])
    dst_p = jnp.concatenate([dst, jnp.full((padlen,), NP - 1, jnp.int32)])
    dst3 = dst_p.reshape(NTILES, CHUNKS, CW)
    srcA = src_p.reshape(16, AGG_CHUNKS, ACW)
    dstA = dst_p.reshape(16, AGG_CHUNKS, ACW)

    degp = _deg_call(dst3)
    deg_col = (degp[0] + degp[1])[:N].reshape(N, 1)
    h0, h1, normc = _norm_call(deg_col, feat)
    agg0, agg1 = _agg_call(h0, h1, srcA, dstA)
    return _out_call(agg0, agg1, normc,
                     weight[:H], weight[H:], bias.reshape(1, D))


# speedup vs baseline: 1.2768x; 1.2768x over previous
import jax
import jax.numpy as jnp
from jax import lax
from jax.experimental import pallas as pl
from jax.experimental.pallas import tpu as pltpu
from jax.experimental.pallas import tpu_sc as plsc

N = 10000
E = 160000
D = 256
H = 128
NP = 10240
EP = 163840
NTILES = 32
CHUNKS = 40
CW = 128
AGG_CHUNKS = 128
ACW = 80
SLABS = 8
SLAB_CHUNKS = AGG_CHUNKS // SLABS
RING = 3
STRIPE = NP // 16

_mesh = plsc.VectorSubcoreMesh(core_axis_name="c", subcore_axis_name="s")


def _deg_kernel(dst3, degp, idx_v, ones_v, buf_v, deg_sh, sem):
    c = lax.axis_index("c")
    s = lax.axis_index("s")
    w = s * 2 + c

    @pl.loop(0, STRIPE, step=16)
    def _(i):
        buf_v[pl.ds(i, 16)] = jnp.zeros((16,), jnp.float32)

    @pl.loop(0, CW, step=16)
    def _(i):
        ones_v[pl.ds(i, 16)] = jnp.ones((16,), jnp.float32)

    pltpu.sync_copy(buf_v, deg_sh.at[pl.ds(s * STRIPE, STRIPE)])
    pltpu.sync_copy(dst3.at[w], idx_v)
    plsc.subcore_barrier()

    @pl.loop(0, CHUNKS)
    def _(j):
        pltpu.sync_copy(ones_v, deg_sh.at[idx_v.at[j]], add=True)

    plsc.subcore_barrier()
    pltpu.sync_copy(deg_sh.at[pl.ds(s * STRIPE, STRIPE)],
                    degp.at[c, pl.ds(s * STRIPE, STRIPE)])


@jax.jit
def _deg_call(dst3):
    k = pl.kernel(
        _deg_kernel,
        out_type=jax.ShapeDtypeStruct((2, NP), jnp.float32),
        mesh=_mesh,
        scratch_types=[
            pltpu.VMEM((CHUNKS, CW), jnp.int32),
            pltpu.VMEM((CW,), jnp.float32),
            pltpu.VMEM((STRIPE,), jnp.float32),
            pltpu.VMEM_SHARED((NP,), jnp.float32),
            pltpu.SemaphoreType.DMA,
        ],
    )
    return k(dst3)


def _agg_kernel(h0, h1, src3, dst3, agg0, agg1,
                srcv, dstv, rows, zbuf, agg_sh, sem, sem2):
    c = lax.axis_index("c")
    s = lax.axis_index("s")

    @pl.loop(0, 8)
    def _(r):
        @pl.loop(0, H, step=16)
        def _(l2):
            zbuf[r, pl.ds(l2, 16)] = jnp.zeros((16,), jnp.float32)

    @pl.loop(0, STRIPE, step=8)
    def _(r):
        pltpu.sync_copy(zbuf, agg_sh.at[pl.ds(s * STRIPE + r, 8)])

    plsc.subcore_barrier()

    def run(h_half, out_half):
        @pl.loop(0, SLABS)
        def _(t):
            pltpu.sync_copy(src3.at[s, pl.ds(t * SLAB_CHUNKS, SLAB_CHUNKS)],
                            srcv)
            pltpu.sync_copy(dst3.at[s, pl.ds(t * SLAB_CHUNKS, SLAB_CHUNKS)],
                            dstv)

            @pl.loop(0, SLAB_CHUNKS + 1)
            def _(j):
                @pl.when(j < SLAB_CHUNKS)
                def _():
                    @pl.when(j >= RING)
                    def _():
                        pltpu.make_async_copy(rows.at[0],
                                              agg_sh.at[pl.ds(0, ACW)],
                                              sem2).wait()

                    pltpu.async_copy(h_half.at[srcv.at[j]],
                                     rows.at[lax.rem(j, RING)], sem)

                @pl.when(j > 0)
                def _():
                    pltpu.make_async_copy(h_half.at[pl.ds(0, ACW)],
                                          rows.at[0], sem).wait()
                    pltpu.async_copy(rows.at[lax.rem(j + RING - 1, RING)],
                                     agg_sh.at[dstv.at[j - 1]], sem2,
                                     add=True)

            @pl.loop(0, RING)
            def _(d):
                pltpu.make_async_copy(rows.at[0], agg_sh.at[pl.ds(0, ACW)],
                                      sem2).wait()

        plsc.subcore_barrier()
        pltpu.sync_copy(agg_sh.at[pl.ds(s * STRIPE, STRIPE)],
                        out_half.at[pl.ds(s * STRIPE, STRIPE)])

    @pl.when(c == 0)
    def _():
        run(h0, agg0)

    @pl.when(c == 1)
    def _():
        run(h1, agg1)


@jax.jit
def _agg_call(h0, h1, src3, dst3):
    k = pl.kernel(
        _agg_kernel,
        out_type=(jax.ShapeDtypeStruct((NP, H), jnp.float32),
                  jax.ShapeDtypeStruct((NP, H), jnp.float32)),
        mesh=_mesh,
        scratch_types=[
            pltpu.VMEM((SLAB_CHUNKS, ACW), jnp.int32),
            pltpu.VMEM((SLAB_CHUNKS, ACW), jnp.int32),
            pltpu.VMEM((RING, ACW, H), jnp.float32),
            pltpu.VMEM((8, H), jnp.float32),
            pltpu.VMEM_SHARED((NP, H), jnp.float32),
            pltpu.SemaphoreType.DMA,
            pltpu.SemaphoreType.DMA,
        ],
    )
    return k(h0, h1, src3, dst3)


def _norm_kernel(deg_ref, feat_ref, h0_ref, h1_ref, norm_ref):
    deg = deg_ref[...]
    norm = jnp.where(deg > 0.0,
                     lax.rsqrt(jnp.where(deg > 0.0, deg, 1.0)), 0.0)
    h = feat_ref[...] * norm
    h0_ref[...] = h[:, :H]
    h1_ref[...] = h[:, H:]
    norm_ref[...] = norm


@jax.jit
def _norm_call(deg_col, feat):
    blk = 1000
    return pl.pallas_call(
        _norm_kernel,
        grid=(N // blk,),
        in_specs=[
            pl.BlockSpec((blk, 1), lambda i: (i, 0)),
            pl.BlockSpec((blk, D), lambda i: (i, 0)),
        ],
        out_specs=[
            pl.BlockSpec((blk, H), lambda i: (i, 0)),
            pl.BlockSpec((blk, H), lambda i: (i, 0)),
            pl.BlockSpec((blk, 1), lambda i: (i, 0)),
        ],
        out_shape=[
            jax.ShapeDtypeStruct((N, H), jnp.float32),
            jax.ShapeDtypeStruct((N, H), jnp.float32),
            jax.ShapeDtypeStruct((N, 1), jnp.float32),
        ],
    )(deg_col, feat)


def _out_kernel(a0_ref, a1_ref, n_ref, w0_ref, w1_ref, b_ref, o_ref):
    r = jnp.dot(a0_ref[...], w0_ref[...],
                preferred_element_type=jnp.float32,
                precision=lax.Precision.HIGHEST)
    r = r + jnp.dot(a1_ref[...], w1_ref[...],
                    preferred_element_type=jnp.float32,
                    precision=lax.Precision.HIGHEST)
    o_ref[...] = r * n_ref[...] + b_ref[...]


@jax.jit
def _out_call(agg0, agg1, normc, w0, w1, bias2):
    blk = 1000
    return pl.pallas_call(
        _out_kernel,
        grid=(N // blk,),
        in_specs=[
            pl.BlockSpec((blk, H), lambda i: (i, 0)),
            pl.BlockSpec((blk, H), lambda i: (i, 0)),
            pl.BlockSpec((blk, 1), lambda i: (i, 0)),
            pl.BlockSpec((H, D), lambda i: (0, 0)),
            pl.BlockSpec((H, D), lambda i: (0, 0)),
            pl.BlockSpec((1, D), lambda i: (0, 0)),
        ],
        out_specs=pl.BlockSpec((blk, D), lambda i: (i, 0)),
        out_shape=jax.ShapeDtypeStruct((N, D), jnp.float32),
    )(agg0, agg1, normc, w0, w1, bias2)


def kernel(feat, edge_index, weight, bias):
    src = edge_index[0]
    dst = edge_index[1]
    padlen = EP - E
    src_p = jnp.concatenate([src, jnp.zeros((padlen,), jnp.int32)])
    dst_p = jnp.concatenate([dst, jnp.full((padlen,), NP - 1, jnp.int32)])
    dst3 = dst_p.reshape(NTILES, CHUNKS, CW)
    srcA = src_p.reshape(16, AGG_CHUNKS, ACW)
    dstA = dst_p.reshape(16, AGG_CHUNKS, ACW)

    degp = _deg_call(dst3)
    deg_col = (degp[0] + degp[1])[:N].reshape(N, 1)
    h0, h1, normc = _norm_call(deg_col, feat)
    agg0, agg1 = _agg_call(h0, h1, srcA, dstA)
    return _out_call(agg0, agg1, normc,
                     weight[:H], weight[H:], bias.reshape(1, D))
